# baseline (speedup 1.0000x reference)
import functools

import jax
import jax.numpy as jnp
from jax import lax
from jax.experimental import pallas as pl
from jax.experimental.pallas import tpu as pltpu
from jax.experimental.pallas import tpu_sc as plsc

B = 16384
D = 128
NC = 2
NS = 16
L = 16
NW = NC * NS
RPW = B // NW
SIZES = (64, 160, 160, 128)
OFFS = (0, 64, 224, 384)
CHMAX = max(SIZES)
NCHUNK = len(SIZES)

_mesh = plsc.VectorSubcoreMesh(core_axis_name="c", subcore_axis_name="s")


@functools.partial(
    pl.kernel,
    mesh=_mesh,
    out_type=jax.ShapeDtypeStruct((B,), jnp.float32),
    compiler_params=pltpu.CompilerParams(needs_layout_passes=False),
    scratch_types=[
        pltpu.VMEM((RPW,), jnp.int32),
        pltpu.VMEM((RPW,), jnp.int32),
        pltpu.VMEM((CHMAX, D), jnp.float32),
        pltpu.VMEM((CHMAX, D), jnp.float32),
        pltpu.VMEM((CHMAX, D), jnp.float32),
        pltpu.VMEM((CHMAX, D), jnp.float32),
        pltpu.VMEM((RPW,), jnp.float32),
        pltpu.VMEM((L * L,), jnp.float32),
        pltpu.SemaphoreType.DMA,
        pltpu.SemaphoreType.DMA,
        pltpu.SemaphoreType.DMA,
        pltpu.SemaphoreType.DMA,
        pltpu.SemaphoreType.DMA,
        pltpu.SemaphoreType.DMA,
        pltpu.SemaphoreType.DMA,
    ],
)
def _acf_scores(uid_hbm, iid_hbm, utab_hbm, itab_hbm, out_hbm,
                idx_u, idx_i, u0, i0, u1, i1, out_v, tpose,
                su0, si0, su1, si1, sxu, sxi, so):
    wid = lax.axis_index("s") * NC + lax.axis_index("c")
    base = wid * RPW

    CH0 = SIZES[0]
    ca_u = pltpu.async_copy(uid_hbm.at[pl.ds(base, CH0)],
                            idx_u.at[pl.ds(0, CH0)], su0)
    ca_i = pltpu.async_copy(iid_hbm.at[pl.ds(base, CH0)],
                            idx_i.at[pl.ds(0, CH0)], si0)
    cb_u = pltpu.async_copy(uid_hbm.at[pl.ds(base + CH0, RPW - CH0)],
                            idx_u.at[pl.ds(CH0, RPW - CH0)], sxu)
    cb_i = pltpu.async_copy(iid_hbm.at[pl.ds(base + CH0, RPW - CH0)],
                            idx_i.at[pl.ds(CH0, RPW - CH0)], sxi)

    bufs = [(u0, i0, su0, si0), (u1, i1, su1, si1)]
    col0 = lax.iota(jnp.int32, L) * L

    def start(c):
        bu, bi, su, si = bufs[c % 2]
        off, sz = OFFS[c], SIZES[c]
        cu = pltpu.async_copy(utab_hbm.at[idx_u.at[pl.ds(off, sz)]],
                              bu.at[pl.ds(0, sz)], su)
        ci = pltpu.async_copy(itab_hbm.at[idx_i.at[pl.ds(off, sz)]],
                              bi.at[pl.ds(0, sz)], si)
        return cu, ci

    def compute(c):
        bu, bi, _, _ = bufs[c % 2]
        off = OFFS[c]

        def group_body(g, carry):
            for r in range(L):
                row = g * L + r
                a = [bu[row, pl.ds(k * L, L)] * bi[row, pl.ds(k * L, L)]
                     for k in range(4)]
                for k in range(4, D // L):
                    a[k % 4] = a[k % 4] + bu[row, pl.ds(k * L, L)] * bi[row, pl.ds(k * L, L)]
                plsc.store_scatter(tpose, [col0 + r], (a[0] + a[1]) + (a[2] + a[3]))
            t = [tpose[pl.ds(d * L, L)] for d in range(L)]
            while len(t) > 1:
                t = [t[2 * j] + t[2 * j + 1] for j in range(len(t) // 2)]
            out_v[pl.ds(off + g * L, L)] = t[0]
            return carry

        lax.fori_loop(0, SIZES[c] // L, group_body, 0)

    out_cps = []

    def out_copy(c):
        off, sz = OFFS[c], SIZES[c]
        out_cps.append(pltpu.async_copy(
            out_v.at[pl.ds(off, sz)],
            out_hbm.at[pl.ds(base + off, sz)], so))

    ca_u.wait()
    ca_i.wait()
    cp = [None, None]
    cp[0] = start(0)
    cb_u.wait()
    cb_i.wait()
    for c in range(NCHUNK):
        if c + 1 < NCHUNK:
            cp[(c + 1) % 2] = start(c + 1)
        cu, ci = cp[c % 2]
        cu.wait()
        ci.wait()
        compute(c)
        out_copy(c)

    for oc in out_cps:
        oc.wait()


def kernel(user_ids, item_ids, user_table, item_table):
    return _acf_scores(user_ids.astype(jnp.int32), item_ids.astype(jnp.int32),
                       user_table, item_table)


# speedup vs baseline: 1.0175x; 1.0175x over previous
import functools

import jax
import jax.numpy as jnp
from jax import lax
from jax.experimental import pallas as pl
from jax.experimental.pallas import tpu as pltpu
from jax.experimental.pallas import tpu_sc as plsc

B = 16384
D = 128
NC = 2
NS = 16
L = 16
NW = NC * NS
RPW = B // NW
SIZES = (64, 160, 160, 128)
OFFS = (0, 64, 224, 384)
CHMAX = max(SIZES)
NCHUNK = len(SIZES)

_mesh = plsc.VectorSubcoreMesh(core_axis_name="c", subcore_axis_name="s")


@functools.partial(
    pl.kernel,
    mesh=_mesh,
    out_type=jax.ShapeDtypeStruct((B,), jnp.float32),
    compiler_params=pltpu.CompilerParams(needs_layout_passes=False),
    scratch_types=[
        pltpu.VMEM((RPW,), jnp.int32),
        pltpu.VMEM((RPW,), jnp.int32),
        pltpu.VMEM((CHMAX, D), jnp.float32),
        pltpu.VMEM((CHMAX, D), jnp.float32),
        pltpu.VMEM((CHMAX, D), jnp.float32),
        pltpu.VMEM((CHMAX, D), jnp.float32),
        pltpu.VMEM((RPW,), jnp.float32),
        pltpu.VMEM((L * L,), jnp.float32),
        pltpu.SemaphoreType.DMA,
        pltpu.SemaphoreType.DMA,
        pltpu.SemaphoreType.DMA,
        pltpu.SemaphoreType.DMA,
        pltpu.SemaphoreType.DMA,
        pltpu.SemaphoreType.DMA,
        pltpu.SemaphoreType.DMA,
    ],
)
def _acf_scores(uid_hbm, iid_hbm, utab_hbm, itab_hbm, out_hbm,
                idx_u, idx_i, u0, i0, u1, i1, out_v, tpose,
                su0, si0, su1, si1, sxu, sxi, so):
    wid = lax.axis_index("s") * NC + lax.axis_index("c")
    base = wid * RPW

    CH0 = SIZES[0]
    ca_u = pltpu.async_copy(uid_hbm.at[pl.ds(base, CH0)],
                            idx_u.at[pl.ds(0, CH0)], su0)
    ca_i = pltpu.async_copy(iid_hbm.at[pl.ds(base, CH0)],
                            idx_i.at[pl.ds(0, CH0)], si0)
    cb_u = pltpu.async_copy(uid_hbm.at[pl.ds(base + CH0, RPW - CH0)],
                            idx_u.at[pl.ds(CH0, RPW - CH0)], sxu)
    cb_i = pltpu.async_copy(iid_hbm.at[pl.ds(base + CH0, RPW - CH0)],
                            idx_i.at[pl.ds(CH0, RPW - CH0)], sxi)

    bufs = [(u0, i0, su0, si0), (u1, i1, su1, si1)]
    col0 = lax.iota(jnp.int32, L) * L

    def start(c):
        bu, bi, su, si = bufs[c % 2]
        off, sz = OFFS[c], SIZES[c]
        cu = pltpu.async_copy(utab_hbm.at[idx_u.at[pl.ds(off, sz)]],
                              bu.at[pl.ds(0, sz)], su)
        ci = pltpu.async_copy(itab_hbm.at[idx_i.at[pl.ds(off, sz)]],
                              bi.at[pl.ds(0, sz)], si)
        return cu, ci

    def compute(c):
        bu, bi, _, _ = bufs[c % 2]
        off = OFFS[c]

        def group_body(g, carry):
            for r in range(L):
                row = g * L + r
                a0 = bu[row, pl.ds(0, L)] * bi[row, pl.ds(0, L)]
                a1 = bu[row, pl.ds(L, L)] * bi[row, pl.ds(L, L)]
                for k in range(2, D // L, 2):
                    a0 = a0 + bu[row, pl.ds(k * L, L)] * bi[row, pl.ds(k * L, L)]
                    a1 = a1 + bu[row, pl.ds((k + 1) * L, L)] * bi[row, pl.ds((k + 1) * L, L)]
                plsc.store_scatter(tpose, [col0 + r], a0 + a1)
            t = [tpose[pl.ds(d * L, L)] for d in range(L)]
            while len(t) > 1:
                t = [t[2 * j] + t[2 * j + 1] for j in range(len(t) // 2)]
            out_v[pl.ds(off + g * L, L)] = t[0]
            return carry

        lax.fori_loop(0, SIZES[c] // L, group_body, 0)

    out_cps = []

    def out_copy(c):
        off, sz = OFFS[c], SIZES[c]
        out_cps.append(pltpu.async_copy(
            out_v.at[pl.ds(off, sz)],
            out_hbm.at[pl.ds(base + off, sz)], so))

    ca_u.wait()
    ca_i.wait()
    cp = [None, None]
    cp[0] = start(0)
    cb_u.wait()
    cb_i.wait()
    for c in range(NCHUNK):
        if c + 1 < NCHUNK:
            cp[(c + 1) % 2] = start(c + 1)
        cu, ci = cp[c % 2]
        cu.wait()
        ci.wait()
        compute(c)
        out_copy(c)

    for oc in out_cps:
        oc.wait()


def kernel(user_ids, item_ids, user_table, item_table):
    return _acf_scores(user_ids.astype(jnp.int32), item_ids.astype(jnp.int32),
                       user_table, item_table)
